# baseline (speedup 1.0000x reference)
import jax
import jax.numpy as jnp
from jax import lax
from jax.experimental import pallas as pl
from jax.experimental.pallas import tpu as pltpu
from jax.experimental.pallas import tpu_sc as plsc

K = 128
BATCH = 16384

NC = 2
NS = 16
NW = NC * NS
R = BATCH // NW
C = 64
NCHUNK = R // C


NBUF = 6


def _mf_body(user_hbm, item_hbm, ut_hbm, it_hbm, w_hbm, out_hbm,
             idx_u, idx_i, u_bufs, i_bufs, w_v, part_v, out_v,
             sem_idx, *sems):
    wid = lax.axis_index("s") * NC + lax.axis_index("c")
    base = wid * R

    idx_copies = []
    for ci in range(NCHUNK):
        idx_copies.append(pltpu.async_copy(
            user_hbm.at[pl.ds(base + ci * C, C)], idx_u.at[ci], sem_idx))
        idx_copies.append(pltpu.async_copy(
            item_hbm.at[pl.ds(base + ci * C, C)], idx_i.at[ci], sem_idx))
    pltpu.sync_copy(w_hbm, w_v)
    for cp in idx_copies:
        cp.wait()

    w_regs = [w_v[pl.ds(16 * j, 16)] for j in range(K // 16)]
    lanes = lax.iota(jnp.int32, 16)
    perms = [lanes ^ (8 >> d) for d in range(4)]
    masks = [(lanes & (8 >> d)) == 0 for d in range(4)]
    bitrev = [0, 8, 4, 12, 2, 10, 6, 14, 1, 9, 5, 13, 3, 11, 7, 15]
    sems = list(sems)

    def issue(ci):
        buf = ci % NBUF
        sem = sems[buf]
        return (pltpu.async_copy(ut_hbm.at[idx_u.at[ci]], u_bufs.at[buf], sem),
                pltpu.async_copy(it_hbm.at[idx_i.at[ci]], i_bufs.at[buf], sem))

    inflight = [issue(ci) for ci in range(NBUF - 1)]
    for ci in range(NCHUNK):
        for cp in inflight.pop(0):
            cp.wait()
        if ci + NBUF - 1 < NCHUNK:
            inflight.append(issue(ci + NBUF - 1))
        buf = ci % NBUF
        u_buf = u_bufs.at[buf]
        i_buf = i_bufs.at[buf]

        @plsc.parallel_loop(0, C, step=2)
        def _phase_a(r2):
            for rr in range(2):
                r = r2 + rr
                acc0 = (u_buf[r, pl.ds(0, 16)] * i_buf[r, pl.ds(0, 16)]
                        * w_regs[0])
                acc1 = (u_buf[r, pl.ds(16, 16)] * i_buf[r, pl.ds(16, 16)]
                        * w_regs[1])
                for j in range(2, K // 16, 2):
                    acc0 = acc0 + (u_buf[r, pl.ds(16 * j, 16)]
                                   * i_buf[r, pl.ds(16 * j, 16)] * w_regs[j])
                    acc1 = acc1 + (u_buf[r, pl.ds(16 * (j + 1), 16)]
                                   * i_buf[r, pl.ds(16 * (j + 1), 16)]
                                   * w_regs[j + 1])
                part_v[pl.ds(r * 16, 16)] = acc0 + acc1

        @plsc.parallel_loop(0, C // 16, step=1)
        def _phase_b(g):
            def merge(a, b, d):
                return (jnp.where(masks[d], a,
                                  jnp.take_along_axis(b, perms[d], axis=0))
                        + jnp.where(masks[d],
                                    jnp.take_along_axis(a, perms[d], axis=0),
                                    b))

            vs = [part_v[pl.ds((g * 16 + bitrev[i]) * 16, 16)]
                  for i in range(16)]
            for d in range(4):
                vs = [merge(vs[2 * p], vs[2 * p + 1], d)
                      for p in range(len(vs) // 2)]
            out_v[pl.ds(ci * C + g * 16, 16)] = vs[0]

    pltpu.sync_copy(out_v, out_hbm.at[pl.ds(base, R)])


@jax.jit
def _mf(user, item, user_table, item_table, w):
    mesh = plsc.VectorSubcoreMesh(core_axis_name="c", subcore_axis_name="s")
    f = pl.kernel(
        _mf_body,
        out_type=jax.ShapeDtypeStruct((BATCH,), jnp.float32),
        mesh=mesh,
        scratch_types=[
            pltpu.VMEM((NCHUNK, C), jnp.int32),
            pltpu.VMEM((NCHUNK, C), jnp.int32),
            pltpu.VMEM((NBUF, C, K), jnp.float32),
            pltpu.VMEM((NBUF, C, K), jnp.float32),
            pltpu.VMEM((K,), jnp.float32),
            pltpu.VMEM((C * 16,), jnp.float32),
            pltpu.VMEM((R,), jnp.float32),
        ] + [pltpu.SemaphoreType.DMA] * (1 + NBUF),
    )
    return f(user, item, user_table, item_table, w)


def kernel(user, item, user_table, item_table, W):
    return _mf(user, item, user_table, item_table, W.reshape(K))


# speedup vs baseline: 1.1089x; 1.1089x over previous
import jax
import jax.numpy as jnp
from jax import lax
from jax.experimental import pallas as pl
from jax.experimental.pallas import tpu as pltpu
from jax.experimental.pallas import tpu_sc as plsc

K = 128
BATCH = 16384

NC = 2
NS = 16
NW = NC * NS
R = BATCH // NW
C = 128
NCHUNK = R // C


NBUF = 3


def _mf_body(user_hbm, item_hbm, ut_hbm, it_hbm, w_hbm, out_hbm,
             idx_u, idx_i, u_bufs, i_bufs, w_v, part_v, out_v,
             sem_idx, *sems):
    wid = lax.axis_index("s") * NC + lax.axis_index("c")
    base = wid * R

    idx_copies = []
    for ci in range(NCHUNK):
        idx_copies.append(pltpu.async_copy(
            user_hbm.at[pl.ds(base + ci * C, C)], idx_u.at[ci], sem_idx))
        idx_copies.append(pltpu.async_copy(
            item_hbm.at[pl.ds(base + ci * C, C)], idx_i.at[ci], sem_idx))
    pltpu.sync_copy(w_hbm, w_v)
    for cp in idx_copies:
        cp.wait()

    w_regs = [w_v[pl.ds(16 * j, 16)] for j in range(K // 16)]
    lanes = lax.iota(jnp.int32, 16)
    perms = [lanes ^ (8 >> d) for d in range(4)]
    masks = [(lanes & (8 >> d)) == 0 for d in range(4)]
    bitrev = [0, 8, 4, 12, 2, 10, 6, 14, 1, 9, 5, 13, 3, 11, 7, 15]
    sems = list(sems)

    def issue(ci):
        buf = ci % NBUF
        sem = sems[buf]
        return (pltpu.async_copy(ut_hbm.at[idx_u.at[ci]], u_bufs.at[buf], sem),
                pltpu.async_copy(it_hbm.at[idx_i.at[ci]], i_bufs.at[buf], sem))

    inflight = [issue(ci) for ci in range(NBUF - 1)]
    for ci in range(NCHUNK):
        for cp in inflight.pop(0):
            cp.wait()
        if ci + NBUF - 1 < NCHUNK:
            inflight.append(issue(ci + NBUF - 1))
        buf = ci % NBUF
        u_buf = u_bufs.at[buf]
        i_buf = i_bufs.at[buf]

        @plsc.parallel_loop(0, C, step=2)
        def _phase_a(r2):
            for rr in range(2):
                r = r2 + rr
                acc0 = (u_buf[r, pl.ds(0, 16)] * i_buf[r, pl.ds(0, 16)]
                        * w_regs[0])
                acc1 = (u_buf[r, pl.ds(16, 16)] * i_buf[r, pl.ds(16, 16)]
                        * w_regs[1])
                for j in range(2, K // 16, 2):
                    acc0 = acc0 + (u_buf[r, pl.ds(16 * j, 16)]
                                   * i_buf[r, pl.ds(16 * j, 16)] * w_regs[j])
                    acc1 = acc1 + (u_buf[r, pl.ds(16 * (j + 1), 16)]
                                   * i_buf[r, pl.ds(16 * (j + 1), 16)]
                                   * w_regs[j + 1])
                part_v[pl.ds(r * 16, 16)] = acc0 + acc1

        @plsc.parallel_loop(0, C // 16, step=1)
        def _phase_b(g):
            def merge(a, b, d):
                return (jnp.where(masks[d], a,
                                  jnp.take_along_axis(b, perms[d], axis=0))
                        + jnp.where(masks[d],
                                    jnp.take_along_axis(a, perms[d], axis=0),
                                    b))

            vs = [part_v[pl.ds((g * 16 + bitrev[i]) * 16, 16)]
                  for i in range(16)]
            for d in range(4):
                vs = [merge(vs[2 * p], vs[2 * p + 1], d)
                      for p in range(len(vs) // 2)]
            out_v[pl.ds(ci * C + g * 16, 16)] = vs[0]

    pltpu.sync_copy(out_v, out_hbm.at[pl.ds(base, R)])


@jax.jit
def _mf(user, item, user_table, item_table, w):
    mesh = plsc.VectorSubcoreMesh(core_axis_name="c", subcore_axis_name="s")
    f = pl.kernel(
        _mf_body,
        out_type=jax.ShapeDtypeStruct((BATCH,), jnp.float32),
        mesh=mesh,
        scratch_types=[
            pltpu.VMEM((NCHUNK, C), jnp.int32),
            pltpu.VMEM((NCHUNK, C), jnp.int32),
            pltpu.VMEM((NBUF, C, K), jnp.float32),
            pltpu.VMEM((NBUF, C, K), jnp.float32),
            pltpu.VMEM((K,), jnp.float32),
            pltpu.VMEM((C * 16,), jnp.float32),
            pltpu.VMEM((R,), jnp.float32),
        ] + [pltpu.SemaphoreType.DMA] * (1 + NBUF),
    )
    return f(user, item, user_table, item_table, w)


def kernel(user, item, user_table, item_table, W):
    return _mf(user, item, user_table, item_table, W.reshape(K))
